# Initial kernel scaffold; baseline (speedup 1.0000x reference)
import jax
import jax.numpy as jnp
from jax.experimental import pallas as pl


def kernel(x, edge_index, adj_weight, dist_vec, W0, W1):
    raise NotImplementedError("write your pallas kernel here")


# speedup vs baseline: 10.9604x; 10.9604x over previous
import functools

import jax
import jax.numpy as jnp
from jax import lax
from jax.experimental import pallas as pl
from jax.experimental.pallas import tpu as pltpu
from jax.experimental.pallas import tpu_sc as plsc

N = 10000
D = 128
E = 320000

NP = 10240
EPAD = 327680
NROW = EPAD // 128
NBT = 80
NSC = 5

_mesh = plsc.VectorSubcoreMesh(core_axis_name="c", subcore_axis_name="s")


def _rsqrt16(t):
    d = jnp.maximum(t, 1.0)
    i = plsc.bitcast(d, jnp.int32)
    i = jnp.int32(0x5F3759DF) - (i >> 1)
    y = plsc.bitcast(i, jnp.float32)
    for _ in range(3):
        y = y * (1.5 - 0.5 * d * y * y)
    return jnp.where(t > 0.5, y, 0.0)


@functools.partial(
    pl.kernel,
    out_type=jax.ShapeDtypeStruct((2 * NP, D), jnp.float32),
    mesh=_mesh,
    compiler_params=pltpu.CompilerParams(needs_layout_passes=False),
    scratch_types=[
        pltpu.VMEM((16, 128), jnp.int32),
        pltpu.VMEM((16, 128), jnp.int32),
        pltpu.VMEM((16, 128), jnp.float32),
        pltpu.VMEM((16, 128), jnp.float32),
        pltpu.VMEM((16, 128), jnp.float32),
        pltpu.VMEM((16, 128), jnp.float32),
        pltpu.VMEM((640,), jnp.float32),
        pltpu.VMEM((2, 128, 128), jnp.float32),
        pltpu.VMEM_SHARED((NP, D), jnp.float32),
        pltpu.VMEM_SHARED((NP,), jnp.float32),
        pltpu.VMEM_SHARED((NP,), jnp.float32),
        pltpu.SemaphoreType.DMA,
        pltpu.SemaphoreType.DMA,
    ],
)
def _sc_spmm(x_h, col_h, row_h, dist_h, part_h,
             colb, rowb, distb, maskb, valsb, dinvg, dtmp, gbuf,
             acc, deg_sh, dinv_sh, sem0, sem1):
    c = lax.axis_index("c")
    s = lax.axis_index("s")
    w = c * 16 + s
    w2 = (1 - c) * 16 + s
    zero16 = jnp.zeros((16,), jnp.float32)
    iota16 = lax.broadcasted_iota(jnp.int32, (16,), 0)

    def _zg(j, _):
        for q in range(8):
            gbuf[0, j, pl.ds(q * 16, 16)] = zero16
        return 0
    lax.fori_loop(0, 128, _zg, 0)

    def _zt(j, _):
        dtmp[pl.ds(j * 16, 16)] = zero16
        return 0
    lax.fori_loop(0, 40, _zt, 0)

    for j in range(5):
        pltpu.sync_copy(gbuf.at[0], acc.at[pl.ds(s * 640 + j * 128, 128)])
    pltpu.sync_copy(dtmp, deg_sh.at[pl.ds(s * 640, 640)])
    plsc.subcore_barrier()

    def _deg_chunk(base_row):
        pltpu.sync_copy(col_h.at[pl.ds(base_row, 16)], colb)

        def _mk(k, _):
            rid = (base_row + k) * 128
            for q in range(8):
                ids = rid + q * 16 + iota16
                maskb[k, pl.ds(q * 16, 16)] = jnp.where(ids < E, 1.0, 0.0)
            return 0
        lax.fori_loop(0, 16, _mk, 0)

        def _fire(k, _):
            pltpu.async_copy(maskb.at[k], deg_sh.at[colb.at[k]], sem1,
                             add=True)
            return 0
        lax.fori_loop(0, 16, _fire, 0)

        def _drain(k, _):
            pltpu.make_async_copy(
                maskb.at[k], deg_sh.at[colb.at[k]], sem1).wait()
            return 0
        lax.fori_loop(0, 16, _drain, 0)

    def _deg_outer(i, _):
        jj = i // 5
        sc_i = i - jj * 5
        base = jnp.where(jj == 0, w, w2) * NBT + sc_i * 16
        _deg_chunk(base)
        return 0
    lax.fori_loop(0, 10, _deg_outer, 0)
    plsc.subcore_barrier()

    pltpu.sync_copy(deg_sh.at[pl.ds(s * 640, 640)], dtmp)

    def _dr(j, _):
        sl = pl.ds(j * 16, 16)
        dtmp[sl] = _rsqrt16(dtmp[sl])
        return 0
    lax.fori_loop(0, 40, _dr, 0)
    pltpu.sync_copy(dtmp, dinv_sh.at[pl.ds(s * 640, 640)])
    plsc.subcore_barrier()

    def _scale_rows(b, get_vec):
        def _s16(t, _):
            vv = get_vec(t)
            for r in range(16):
                scl = lax.broadcast(vv[r], (16,))
                for q in range(8):
                    sl = pl.ds(q * 16, 16)
                    gbuf[b, t * 16 + r, sl] = gbuf[b, t * 16 + r, sl] * scl
            return 0
        lax.fori_loop(0, 8, _s16, 0)

    def _chunk(sc_i, _):
        base_row = w * NBT + sc_i * 16
        pltpu.sync_copy(col_h.at[pl.ds(base_row, 16)], colb)
        pltpu.sync_copy(row_h.at[pl.ds(base_row, 16)], rowb)
        pltpu.sync_copy(dist_h.at[pl.ds(base_row, 16)], distb)

        def _gfire(k, _):
            pltpu.async_copy(dinv_sh.at[colb.at[k]], dinvg.at[k], sem1)
            return 0
        lax.fori_loop(0, 16, _gfire, 0)

        def _gdrain(k, _):
            pltpu.make_async_copy(
                dinv_sh.at[colb.at[k]], dinvg.at[k], sem1).wait()
            return 0
        lax.fori_loop(0, 16, _gdrain, 0)

        def _vc(k, _):
            rid = (base_row + k) * 128
            for q in range(8):
                sl = pl.ds(q * 16, 16)
                dv = distb[k, sl]
                ids = rid + q * 16 + iota16
                g = jnp.where(ids < E, dinvg[k, sl], 0.0)
                valsb[k, sl] = jnp.exp(-(dv * dv)) * g
            return 0
        lax.fori_loop(0, 16, _vc, 0)

        pltpu.async_copy(x_h.at[colb.at[0]], gbuf.at[0], sem0)

        def _mloop(i, _):
            k0 = 2 * i
            k1 = 2 * i + 1
            pltpu.async_copy(x_h.at[colb.at[k1]], gbuf.at[1], sem1)
            pltpu.make_async_copy(
                x_h.at[colb.at[k0]], gbuf.at[0], sem0).wait()
            _scale_rows(0, lambda t: valsb[k0, pl.ds(t * 16, 16)])
            pltpu.sync_copy(gbuf.at[0], acc.at[rowb.at[k0]], add=True)

            @pl.when(i < 7)
            def _():
                pltpu.async_copy(x_h.at[colb.at[k0 + 2]], gbuf.at[0], sem0)

            pltpu.make_async_copy(
                x_h.at[colb.at[k1]], gbuf.at[1], sem1).wait()
            _scale_rows(1, lambda t: valsb[k1, pl.ds(t * 16, 16)])
            pltpu.sync_copy(gbuf.at[1], acc.at[rowb.at[k1]], add=True)
            return 0
        lax.fori_loop(0, 8, _mloop, 0)
        return 0
    lax.fori_loop(0, NSC, _chunk, 0)
    plsc.subcore_barrier()

    def _wb(j, _):
        base = s * 640 + j * 128
        pltpu.sync_copy(acc.at[pl.ds(base, 128)], gbuf.at[0])
        pltpu.sync_copy(dinv_sh.at[pl.ds(base, 128)], dtmp.at[pl.ds(0, 128)])
        _scale_rows(0, lambda t: dtmp[pl.ds(t * 16, 16)])
        pltpu.sync_copy(gbuf.at[0], part_h.at[pl.ds(c * NP + base, 128)])
        return 0
    lax.fori_loop(0, 5, _wb, 0)


def _tc_body(p0_ref, p1_ref, x_ref, w0_ref, w1_ref, o_ref):
    side = p0_ref[...] + p1_ref[...]
    bi = x_ref[...] * side
    dn = (((1,), (1,)), ((), ()))
    o_ref[...] = (
        lax.dot_general(side, w0_ref[...], dn, preferred_element_type=jnp.float32)
        + lax.dot_general(bi, w1_ref[...], dn, preferred_element_type=jnp.float32)
    )


@jax.jit
def _run(x, edge_index, dist_vec, W0, W1):
    row = edge_index[0]
    col = edge_index[1]
    pad = EPAD - E
    zi = jnp.zeros((pad,), jnp.int32)
    colp = jnp.concatenate([col, zi]).reshape(NROW, 128)
    rowp = jnp.concatenate([row, zi]).reshape(NROW, 128)
    distp = jnp.concatenate(
        [dist_vec, jnp.zeros((pad,), jnp.float32)]).reshape(NROW, 128)

    part = _sc_spmm(x, colp, rowp, distp)
    p0 = part[:N]
    p1 = part[NP:NP + N]

    blk = 2000
    out = pl.pallas_call(
        _tc_body,
        grid=(N // blk,),
        in_specs=[
            pl.BlockSpec((blk, D), lambda i: (i, 0)),
            pl.BlockSpec((blk, D), lambda i: (i, 0)),
            pl.BlockSpec((blk, D), lambda i: (i, 0)),
            pl.BlockSpec((D, D), lambda i: (0, 0)),
            pl.BlockSpec((D, D), lambda i: (0, 0)),
        ],
        out_specs=pl.BlockSpec((blk, D), lambda i: (i, 0)),
        out_shape=jax.ShapeDtypeStruct((N, D), jnp.float32),
    )(p0, p1, x, W0, W1)
    return out


def kernel(x, edge_index, adj_weight, dist_vec, W0, W1):
    del adj_weight
    return _run(x, edge_index, dist_vec, W0, W1)


# speedup vs baseline: 12.7285x; 1.1613x over previous
import functools

import jax
import jax.numpy as jnp
from jax import lax
from jax.experimental import pallas as pl
from jax.experimental.pallas import tpu as pltpu
from jax.experimental.pallas import tpu_sc as plsc

N = 10000
D = 128
E = 320000

NP = 10240
EPAD = 327680
NROW = EPAD // 128
RPT = NROW // 16
C0_CH = 8
C1_CH = 2

_mesh = plsc.VectorSubcoreMesh(core_axis_name="c", subcore_axis_name="s")


def _rsqrt16(t):
    d = jnp.maximum(t, 1.0)
    i = plsc.bitcast(d, jnp.int32)
    i = jnp.int32(0x5F3759DF) - (i >> 1)
    y = plsc.bitcast(i, jnp.float32)
    for _ in range(3):
        y = y * (1.5 - 0.5 * d * y * y)
    return jnp.where(t > 0.5, y, 0.0)


@functools.partial(
    pl.kernel,
    out_type=jax.ShapeDtypeStruct((2 * NP, D), jnp.float32),
    mesh=_mesh,
    compiler_params=pltpu.CompilerParams(needs_layout_passes=False),
    scratch_types=[
        pltpu.VMEM((16, 128), jnp.int32),
        pltpu.VMEM((16, 128), jnp.int32),
        pltpu.VMEM((16, 128), jnp.float32),
        pltpu.VMEM((16, 128), jnp.float32),
        pltpu.VMEM((16, 128), jnp.float32),
        pltpu.VMEM((640,), jnp.float32),
        pltpu.VMEM((2, 128, 128), jnp.float32),
        pltpu.VMEM_SHARED((NP, D), jnp.float32),
        pltpu.VMEM_SHARED((NP,), jnp.float32),
        pltpu.VMEM_SHARED((NP,), jnp.float32),
        pltpu.HBM((NP, D), jnp.float32),
        pltpu.SemaphoreType.DMA,
        pltpu.SemaphoreType.DMA,
    ],
)
def _sc_spmm(x_h, col_h, row_h, dist_h, part_h,
             colb, rowb, distb, maskb, valsb, dtmp, gbuf,
             acc, deg_sh, dinv_sh, y_h, sem0, sem1):
    c = lax.axis_index("c")
    s = lax.axis_index("s")
    zero16 = jnp.zeros((16,), jnp.float32)
    iota16 = lax.broadcasted_iota(jnp.int32, (16,), 0)

    def _zg(j, _):
        for q in range(8):
            gbuf[0, j, pl.ds(q * 16, 16)] = zero16
        return 0
    lax.fori_loop(0, 128, _zg, 0)

    def _zt(j, _):
        dtmp[pl.ds(j * 16, 16)] = zero16
        return 0
    lax.fori_loop(0, 40, _zt, 0)

    for j in range(5):
        pltpu.sync_copy(gbuf.at[0], acc.at[pl.ds(s * 640 + j * 128, 128)])
    pltpu.sync_copy(dtmp, deg_sh.at[pl.ds(s * 640, 640)])
    plsc.subcore_barrier()

    def _deg_chunk(i, _):
        base_row = s * RPT + i * 16
        pltpu.sync_copy(col_h.at[pl.ds(base_row, 16)], colb)

        def _mk(k, _):
            rid = (base_row + k) * 128
            for q in range(8):
                ids = rid + q * 16 + iota16
                maskb[k, pl.ds(q * 16, 16)] = jnp.where(ids < E, 1.0, 0.0)
            return 0
        lax.fori_loop(0, 16, _mk, 0)

        def _fire(k, _):
            pltpu.async_copy(maskb.at[k], deg_sh.at[colb.at[k]], sem1,
                             add=True)
            return 0
        lax.fori_loop(0, 16, _fire, 0)

        def _drain(k, _):
            pltpu.make_async_copy(
                maskb.at[k], deg_sh.at[colb.at[k]], sem1).wait()
            return 0
        lax.fori_loop(0, 16, _drain, 0)
        return 0
    lax.fori_loop(0, RPT // 16, _deg_chunk, 0)
    plsc.subcore_barrier()

    pltpu.sync_copy(deg_sh.at[pl.ds(s * 640, 640)], dtmp)

    def _dr(j, _):
        sl = pl.ds(j * 16, 16)
        dtmp[sl] = _rsqrt16(dtmp[sl])
        return 0
    lax.fori_loop(0, 40, _dr, 0)
    pltpu.sync_copy(dtmp, dinv_sh.at[pl.ds(s * 640, 640)])
    plsc.subcore_barrier()

    def _scale_rows(b, get_vec):
        def _s16(t, _):
            vv = get_vec(t)
            for r in range(16):
                scl = lax.broadcast(vv[r], (16,))
                for q in range(8):
                    sl = pl.ds(q * 16, 16)
                    gbuf[b, t * 16 + r, sl] = gbuf[b, t * 16 + r, sl] * scl
            return 0
        lax.fori_loop(0, 8, _s16, 0)

    def _ystage(j, _):
        base = s * 640 + j * 128
        pltpu.sync_copy(x_h.at[pl.ds(base, 128)], gbuf.at[0])
        pltpu.sync_copy(dinv_sh.at[pl.ds(base, 128)], dtmp.at[pl.ds(0, 128)])
        _scale_rows(0, lambda t: dtmp[pl.ds(t * 16, 16)])
        pltpu.sync_copy(gbuf.at[0], y_h.at[pl.ds(base, 128)])
        return 0
    lax.fori_loop(0, 5, _ystage, 0)
    plsc.subcore_barrier()

    nch = jnp.where(c == 0, C0_CH, C1_CH)
    cbase = s * RPT + jnp.where(c == 0, 0, C0_CH * 16)

    def _chunk(sc_i, _):
        base_row = cbase + sc_i * 16
        pltpu.sync_copy(col_h.at[pl.ds(base_row, 16)], colb)
        pltpu.sync_copy(row_h.at[pl.ds(base_row, 16)], rowb)
        pltpu.sync_copy(dist_h.at[pl.ds(base_row, 16)], distb)

        def _vc(k, _):
            rid = (base_row + k) * 128
            for q in range(8):
                sl = pl.ds(q * 16, 16)
                dv = distb[k, sl]
                ids = rid + q * 16 + iota16
                wv = jnp.exp(-(dv * dv))
                valsb[k, sl] = jnp.where(ids < E, wv, 0.0)
            return 0
        lax.fori_loop(0, 16, _vc, 0)

        pltpu.async_copy(y_h.at[colb.at[0]], gbuf.at[0], sem0)

        def _mloop(i, _):
            k0 = 2 * i
            k1 = 2 * i + 1
            pltpu.async_copy(y_h.at[colb.at[k1]], gbuf.at[1], sem1)
            pltpu.make_async_copy(
                y_h.at[colb.at[k0]], gbuf.at[0], sem0).wait()
            _scale_rows(0, lambda t: valsb[k0, pl.ds(t * 16, 16)])
            pltpu.sync_copy(gbuf.at[0], acc.at[rowb.at[k0]], add=True)

            @pl.when(i < 7)
            def _():
                pltpu.async_copy(y_h.at[colb.at[k0 + 2]], gbuf.at[0], sem0)

            pltpu.make_async_copy(
                y_h.at[colb.at[k1]], gbuf.at[1], sem1).wait()
            _scale_rows(1, lambda t: valsb[k1, pl.ds(t * 16, 16)])
            pltpu.sync_copy(gbuf.at[1], acc.at[rowb.at[k1]], add=True)
            return 0
        lax.fori_loop(0, 8, _mloop, 0)
        return 0
    lax.fori_loop(0, nch, _chunk, 0)
    plsc.subcore_barrier()

    def _wb(j, _):
        base = s * 640 + j * 128
        pltpu.sync_copy(acc.at[pl.ds(base, 128)], gbuf.at[0])
        pltpu.sync_copy(dinv_sh.at[pl.ds(base, 128)], dtmp.at[pl.ds(0, 128)])
        _scale_rows(0, lambda t: dtmp[pl.ds(t * 16, 16)])
        pltpu.sync_copy(gbuf.at[0], part_h.at[pl.ds(c * NP + base, 128)])
        return 0
    lax.fori_loop(0, 5, _wb, 0)


def _tc_body(p0_ref, p1_ref, x_ref, w0_ref, w1_ref, o_ref):
    side = p0_ref[...] + p1_ref[...]
    bi = x_ref[...] * side
    dn = (((1,), (1,)), ((), ()))
    o_ref[...] = (
        lax.dot_general(side, w0_ref[...], dn, preferred_element_type=jnp.float32)
        + lax.dot_general(bi, w1_ref[...], dn, preferred_element_type=jnp.float32)
    )


@jax.jit
def _run(x, edge_index, dist_vec, W0, W1):
    row = edge_index[0]
    col = edge_index[1]
    pad = EPAD - E
    zi = jnp.zeros((pad,), jnp.int32)
    colp = jnp.concatenate([col, zi]).reshape(NROW, 128)
    rowp = jnp.concatenate([row, zi]).reshape(NROW, 128)
    distp = jnp.concatenate(
        [dist_vec, jnp.zeros((pad,), jnp.float32)]).reshape(NROW, 128)
    xp = jnp.concatenate([x, jnp.zeros((NP - N, D), jnp.float32)])

    part = _sc_spmm(xp, colp, rowp, distp)
    p0 = part[:N]
    p1 = part[NP:NP + N]

    blk = 2000
    out = pl.pallas_call(
        _tc_body,
        grid=(N // blk,),
        in_specs=[
            pl.BlockSpec((blk, D), lambda i: (i, 0)),
            pl.BlockSpec((blk, D), lambda i: (i, 0)),
            pl.BlockSpec((blk, D), lambda i: (i, 0)),
            pl.BlockSpec((D, D), lambda i: (0, 0)),
            pl.BlockSpec((D, D), lambda i: (0, 0)),
        ],
        out_specs=pl.BlockSpec((blk, D), lambda i: (i, 0)),
        out_shape=jax.ShapeDtypeStruct((N, D), jnp.float32),
    )(p0, p1, x, W0, W1)
    return out


def kernel(x, edge_index, adj_weight, dist_vec, W0, W1):
    del adj_weight
    return _run(x, edge_index, dist_vec, W0, W1)
